# baseline (speedup 1.0000x reference)
import functools

import jax
import jax.numpy as jnp
from jax import lax
from jax.experimental import pallas as pl
from jax.experimental.pallas import tpu as pltpu

_NUM_CLASSES = 21
_POS_THR = 0.5
_NEG_THR = 0.1
_TOTAL = 128
_MAX_POS = 32
_N = 5000
_NGT = 50
_BITS_LO_P = 0x3FC00000
_BITS_LO_N = 0x40000000
_STEP0_P = 0x40000
_STEP0_N = 0x60000
_STEP0_J = 313


def _cnt(mask):
    return jnp.sum(mask.astype(jnp.int32), axis=1, keepdims=True)


def _c11(v):
    return jnp.full((1, 1), v, jnp.int32)


def _select_topk2(pkey, kp, nkey, kn, idx):
    bp = lax.bitcast_convert_type(pkey, jnp.int32)
    bn = lax.bitcast_convert_type(nkey, jnp.int32)
    sub = lax.broadcasted_iota(jnp.int32, (16, 1), 0)

    def pick(bits, lo, step, k):
        counts = jnp.sum((bits >= lo + sub * step).astype(jnp.int32),
                         axis=1, keepdims=True)
        return jnp.sum((counts >= k).astype(jnp.int32), axis=0,
                       keepdims=True)

    def tbody(_, c):
        lop, sp, lon, sn = c
        ip = pick(bp, lop, sp, kp)
        iq = pick(bn, lon, sn, kn)
        return (lop + (ip - 1) * sp, (sp + 15) // 16,
                lon + (iq - 1) * sn, (sn + 15) // 16)

    lop, _, lon, _ = lax.fori_loop(
        0, 6, tbody,
        (_c11(_BITS_LO_P), _c11(_STEP0_P), _c11(_BITS_LO_N), _c11(_STEP0_N)),
        unroll=True)
    tp = jnp.where(_cnt(bp >= _BITS_LO_P) >= kp, lop, 0)
    tn = jnp.where(_cnt(bn >= _BITS_LO_N) >= kn, lon, 0)

    eqp = bp == tp
    eqn = bn == tn
    needp = kp - _cnt(bp > tp)
    needn = kn - _cnt(bn > tn)
    wp = jnp.where(eqp, -idx, -(1 << 30))
    wn = jnp.where(eqn, -idx, -(1 << 30))

    def jbody(_, c):
        lp, sp, ln, sn = c
        ip = pick(wp, lp, sp, needp)
        iq = pick(wn, ln, sn, needn)
        return (lp + (ip - 1) * sp, (sp + 15) // 16,
                ln + (iq - 1) * sn, (sn + 15) // 16)

    up, _, un, _ = lax.fori_loop(
        0, 4, jbody,
        (_c11(1 - _N), _c11(_STEP0_J), _c11(1 - _N), _c11(_STEP0_J)),
        unroll=True)

    pos_sel = (bp > tp) | (eqp & (idx <= -up))
    neg_sel = (bn > tn) | (eqn & (idx <= -un))
    return pos_sel, neg_sel


def _roi_kernel(ishape_ref, roist_ref, scoret_ref, bboxt_ref, gtb_ref, gtl_ref,
                cls_ref, reg_ref):
    hf = ishape_ref[0].astype(jnp.float32)
    wf = ishape_ref[1].astype(jnp.float32)

    x1 = jnp.clip(roist_ref[0:1, :], 0.0, wf - 1.0)
    y1 = jnp.clip(roist_ref[1:2, :], 0.0, hf - 1.0)
    x2 = jnp.clip(roist_ref[2:3, :], 0.0, wf - 1.0)
    y2 = jnp.clip(roist_ref[3:4, :], 0.0, hf - 1.0)

    gx1 = gtb_ref[:, 0:1]
    gy1 = gtb_ref[:, 1:2]
    gx2 = gtb_ref[:, 2:3]
    gy2 = gtb_ref[:, 3:4]

    area_r = (x2 - x1) * (y2 - y1)
    area_g = (gx2 - gx1) * (gy2 - gy1)
    ltx = jnp.maximum(gx1, x1)
    lty = jnp.maximum(gy1, y1)
    rbx = jnp.minimum(gx2, x2)
    rby = jnp.minimum(gy2, y2)
    whx = jnp.clip(rbx - ltx, 0.0, None)
    why = jnp.clip(rby - lty, 0.0, None)
    inter = whx * why
    union = area_r + area_g - inter
    iou = inter / jnp.maximum(union, 1e-8)

    max_iou = jnp.max(iou, axis=0, keepdims=True)
    g_iota = lax.broadcasted_iota(jnp.int32, (_NGT, _N), 0)
    am = jnp.min(jnp.where(iou == max_iou, g_iota, _NGT), axis=0,
                 keepdims=True)

    onehot_f = (g_iota == am).astype(jnp.float32)
    gtcat = jnp.concatenate(
        [gtb_ref[:, :], gtl_ref[:, :].astype(jnp.float32)], axis=1)
    mg = lax.dot_general(gtcat, onehot_f, (((0,), (0,)), ((), ())),
                         precision=lax.Precision.HIGHEST)
    mgx1 = mg[0:1, :]
    mgy1 = mg[1:2, :]
    mgx2 = mg[2:3, :]
    mgy2 = mg[3:4, :]
    lab = mg[4:5, :].astype(jnp.int32)

    pos = max_iou >= _POS_THR
    pkey = jnp.where(pos, 1.0 + max_iou, 0.0)
    neg_pref = (max_iou < _POS_THR) & (max_iou >= _NEG_THR)
    neg_back = max_iou < _NEG_THR
    nkey = jnp.where(neg_pref, 3.0 + max_iou,
                     jnp.where(neg_back, 2.0 + max_iou, 0.0))

    npos = _cnt(pos)
    pos_num = jnp.minimum(npos, _MAX_POS)
    k_neg = _TOTAL - pos_num

    idx = lax.broadcasted_iota(jnp.int32, (1, _N), 1)
    pos_sel, neg_sel = _select_topk2(pkey, pos_num, nkey, k_neg, idx)

    scores = scoret_ref[:, :]
    m = jnp.max(scores, axis=0, keepdims=True)
    lse = m + jnp.log(jnp.sum(jnp.exp(scores - m), axis=0, keepdims=True))
    c_iota = lax.broadcasted_iota(jnp.int32, (_NUM_CLASSES, _N), 0)
    logp_lab = jnp.sum(jnp.where(c_iota == lab, scores, 0.0), axis=0,
                       keepdims=True) - lse
    logp0 = scores[0:1, :] - lse
    cls_sum = jnp.sum(jnp.where(pos_sel, -logp_lab, 0.0)
                      + jnp.where(neg_sel, -logp0, 0.0),
                      axis=1, keepdims=True)
    cls_ref[:, :] = cls_sum / float(_TOTAL)

    pw = jnp.maximum(x2 - x1, 1.0)
    ph = jnp.maximum(y2 - y1, 1.0)
    px = x1 + 0.5 * pw
    py = y1 + 0.5 * ph
    gw = jnp.maximum(mgx2 - mgx1, 1.0)
    gh = jnp.maximum(mgy2 - mgy1, 1.0)
    gx = mgx1 + 0.5 * gw
    gy = mgy1 + 0.5 * gh
    tx = (gx - px) / pw
    ty = (gy - py) / ph
    tw = jnp.log(gw / pw)
    th = jnp.log(gh / ph)
    t4 = jnp.concatenate([tx, ty, tw, th], axis=0)
    t84 = jnp.tile(t4, (_NUM_CLASSES, 1))

    preds = bboxt_ref[:, :]
    diff = preds - t84
    abs_d = jnp.abs(diff)
    sl1 = jnp.where(abs_d < 1.0, 0.5 * diff * diff, abs_d - 0.5)
    r_iota = lax.broadcasted_iota(jnp.int32, (4 * _NUM_CLASSES, _N), 0)
    cls_of_row = r_iota // 4
    per_roi = jnp.sum(jnp.where(cls_of_row == lab, sl1, 0.0), axis=0,
                      keepdims=True)
    reg_sum = jnp.sum(jnp.where(pos_sel, per_roi, 0.0), axis=1,
                      keepdims=True)
    reg_ref[:, :] = reg_sum / jnp.maximum(pos_num.astype(jnp.float32), 1.0)


@jax.jit
def _run(ishape, roist, scoret, bboxt, gtb, gtl2):
    out = pl.pallas_call(
        _roi_kernel,
        out_shape=[
            jax.ShapeDtypeStruct((1, 1), jnp.float32),
            jax.ShapeDtypeStruct((1, 1), jnp.float32),
        ],
        in_specs=[
            pl.BlockSpec(memory_space=pltpu.SMEM),
            pl.BlockSpec(memory_space=pltpu.VMEM),
            pl.BlockSpec(memory_space=pltpu.VMEM),
            pl.BlockSpec(memory_space=pltpu.VMEM),
            pl.BlockSpec(memory_space=pltpu.VMEM),
            pl.BlockSpec(memory_space=pltpu.VMEM),
        ],
        out_specs=[
            pl.BlockSpec(memory_space=pltpu.VMEM),
            pl.BlockSpec(memory_space=pltpu.VMEM),
        ],
    )(ishape, roist, scoret, bboxt, gtb, gtl2)
    return out[0][0, 0], out[1][0, 0]


def kernel(image_shape, rois, roi_score, roi_bboxes_txtytwth, gt_bboxes,
           gt_labels):
    gtl2 = gt_labels.astype(jnp.int32).reshape(_NGT, 1)
    return _run(image_shape.astype(jnp.int32),
                rois.astype(jnp.float32).T,
                roi_score.astype(jnp.float32).T,
                roi_bboxes_txtytwth.astype(jnp.float32).T,
                gt_bboxes.astype(jnp.float32), gtl2)


# speedup vs baseline: 1.0146x; 1.0146x over previous
import jax
import jax.numpy as jnp
from jax import lax
from jax.experimental import pallas as pl
from jax.experimental.pallas import tpu as pltpu

_NUM_CLASSES = 21
_POS_THR = 0.5
_NEG_THR = 0.1
_TOTAL = 128
_MAX_POS = 32
_N = 5000
_NGT = 50
_BITS_LO_P = 0x3FC00000
_BITS_LO_N = 0x40000000
_STEP0_P = 0x80000
_STEP0_N = 0xC0000
_STEP0_J = 625


def _cnt(mask):
    return jnp.sum(mask.astype(jnp.int32), axis=1, keepdims=True)


def _c11(v):
    return jnp.full((1, 1), v, jnp.int32)


def _select_topk2(pkey, kp, nkey, kn, idx):
    bp = lax.bitcast_convert_type(pkey, jnp.int32)
    bn = lax.bitcast_convert_type(nkey, jnp.int32)
    sub = lax.broadcasted_iota(jnp.int32, (8, 1), 0)

    def pick(bits, lo, step, k):
        counts = jnp.sum((bits >= lo + sub * step).astype(jnp.int32),
                         axis=1, keepdims=True)
        return jnp.sum((counts >= k).astype(jnp.int32), axis=0,
                       keepdims=True)

    def tbody(_, c):
        lop, sp, lon, sn = c
        ip = pick(bp, lop, sp, kp)
        iq = pick(bn, lon, sn, kn)
        return (lop + (ip - 1) * sp, (sp + 7) // 8,
                lon + (iq - 1) * sn, (sn + 7) // 8)

    lop, _, lon, _ = lax.fori_loop(
        0, 8, tbody,
        (_c11(_BITS_LO_P), _c11(_STEP0_P), _c11(_BITS_LO_N), _c11(_STEP0_N)),
        unroll=True)
    tp = jnp.where(_cnt(bp >= _BITS_LO_P) >= kp, lop, 0)
    tn = jnp.where(_cnt(bn >= _BITS_LO_N) >= kn, lon, 0)

    eqp = bp == tp
    eqn = bn == tn
    needp = kp - _cnt(bp > tp)
    needn = kn - _cnt(bn > tn)
    wp = jnp.where(eqp, -idx, -(1 << 30))
    wn = jnp.where(eqn, -idx, -(1 << 30))

    def jbody(_, c):
        lp, sp, ln, sn = c
        ip = pick(wp, lp, sp, needp)
        iq = pick(wn, ln, sn, needn)
        return (lp + (ip - 1) * sp, (sp + 7) // 8,
                ln + (iq - 1) * sn, (sn + 7) // 8)

    up, _, un, _ = lax.fori_loop(
        0, 5, jbody,
        (_c11(1 - _N), _c11(_STEP0_J), _c11(1 - _N), _c11(_STEP0_J)),
        unroll=True)

    pos_sel = (bp > tp) | (eqp & (idx <= -up))
    neg_sel = (bn > tn) | (eqn & (idx <= -un))
    return pos_sel, neg_sel


def _roi_kernel(ishape_ref, roist_ref, scoret_ref, bboxt_ref, gtb_ref, gtl_ref,
                cls_ref, reg_ref):
    hf = ishape_ref[0].astype(jnp.float32)
    wf = ishape_ref[1].astype(jnp.float32)

    x1 = jnp.clip(roist_ref[0:1, :], 0.0, wf - 1.0)
    y1 = jnp.clip(roist_ref[1:2, :], 0.0, hf - 1.0)
    x2 = jnp.clip(roist_ref[2:3, :], 0.0, wf - 1.0)
    y2 = jnp.clip(roist_ref[3:4, :], 0.0, hf - 1.0)

    gx1 = gtb_ref[:, 0:1]
    gy1 = gtb_ref[:, 1:2]
    gx2 = gtb_ref[:, 2:3]
    gy2 = gtb_ref[:, 3:4]

    area_r = (x2 - x1) * (y2 - y1)
    area_g = (gx2 - gx1) * (gy2 - gy1)
    ltx = jnp.maximum(gx1, x1)
    lty = jnp.maximum(gy1, y1)
    rbx = jnp.minimum(gx2, x2)
    rby = jnp.minimum(gy2, y2)
    whx = jnp.clip(rbx - ltx, 0.0, None)
    why = jnp.clip(rby - lty, 0.0, None)
    inter = whx * why
    union = area_r + area_g - inter
    iou = inter / jnp.maximum(union, 1e-8)

    max_iou = jnp.max(iou, axis=0, keepdims=True)
    g_iota = lax.broadcasted_iota(jnp.int32, (_NGT, _N), 0)
    am = jnp.min(jnp.where(iou == max_iou, g_iota, _NGT), axis=0,
                 keepdims=True)

    onehot_f = (g_iota == am).astype(jnp.float32)
    gtcat = jnp.concatenate(
        [gtb_ref[:, :], gtl_ref[:, :].astype(jnp.float32)], axis=1)
    mg = lax.dot_general(gtcat, onehot_f, (((0,), (0,)), ((), ())),
                         precision=lax.Precision.HIGHEST)
    mgx1 = mg[0:1, :]
    mgy1 = mg[1:2, :]
    mgx2 = mg[2:3, :]
    mgy2 = mg[3:4, :]
    lab = mg[4:5, :].astype(jnp.int32)

    pos = max_iou >= _POS_THR
    pkey = jnp.where(pos, 1.0 + max_iou, 0.0)
    neg_pref = (max_iou < _POS_THR) & (max_iou >= _NEG_THR)
    neg_back = max_iou < _NEG_THR
    nkey = jnp.where(neg_pref, 3.0 + max_iou,
                     jnp.where(neg_back, 2.0 + max_iou, 0.0))

    npos = _cnt(pos)
    pos_num = jnp.minimum(npos, _MAX_POS)
    k_neg = _TOTAL - pos_num

    idx = lax.broadcasted_iota(jnp.int32, (1, _N), 1)
    pos_sel, neg_sel = _select_topk2(pkey, pos_num, nkey, k_neg, idx)

    scores = scoret_ref[:, :]
    m = jnp.max(scores, axis=0, keepdims=True)
    lse = m + jnp.log(jnp.sum(jnp.exp(scores - m), axis=0, keepdims=True))
    c_iota = lax.broadcasted_iota(jnp.int32, (_NUM_CLASSES, _N), 0)
    logp_lab = jnp.sum(jnp.where(c_iota == lab, scores, 0.0), axis=0,
                       keepdims=True) - lse
    logp0 = scores[0:1, :] - lse
    cls_sum = jnp.sum(jnp.where(pos_sel, -logp_lab, 0.0)
                      + jnp.where(neg_sel, -logp0, 0.0),
                      axis=1, keepdims=True)
    cls_ref[:, :] = cls_sum / float(_TOTAL)

    pw = jnp.maximum(x2 - x1, 1.0)
    ph = jnp.maximum(y2 - y1, 1.0)
    px = x1 + 0.5 * pw
    py = y1 + 0.5 * ph
    gw = jnp.maximum(mgx2 - mgx1, 1.0)
    gh = jnp.maximum(mgy2 - mgy1, 1.0)
    gx = mgx1 + 0.5 * gw
    gy = mgy1 + 0.5 * gh
    tx = (gx - px) / pw
    ty = (gy - py) / ph
    tw = jnp.log(gw / pw)
    th = jnp.log(gh / ph)
    t4 = jnp.concatenate([tx, ty, tw, th], axis=0)
    t84 = jnp.tile(t4, (_NUM_CLASSES, 1))

    preds = bboxt_ref[:, :]
    diff = preds - t84
    abs_d = jnp.abs(diff)
    sl1 = jnp.where(abs_d < 1.0, 0.5 * diff * diff, abs_d - 0.5)
    r_iota = lax.broadcasted_iota(jnp.int32, (4 * _NUM_CLASSES, _N), 0)
    cls_of_row = r_iota // 4
    per_roi = jnp.sum(jnp.where(cls_of_row == lab, sl1, 0.0), axis=0,
                      keepdims=True)
    reg_sum = jnp.sum(jnp.where(pos_sel, per_roi, 0.0), axis=1,
                      keepdims=True)
    reg_ref[:, :] = reg_sum / jnp.maximum(pos_num.astype(jnp.float32), 1.0)


@jax.jit
def _run(ishape, roist, scoret, bboxt, gtb, gtl2):
    out = pl.pallas_call(
        _roi_kernel,
        out_shape=[
            jax.ShapeDtypeStruct((1, 1), jnp.float32),
            jax.ShapeDtypeStruct((1, 1), jnp.float32),
        ],
        in_specs=[
            pl.BlockSpec(memory_space=pltpu.SMEM),
            pl.BlockSpec(memory_space=pltpu.VMEM),
            pl.BlockSpec(memory_space=pltpu.VMEM),
            pl.BlockSpec(memory_space=pltpu.VMEM),
            pl.BlockSpec(memory_space=pltpu.VMEM),
            pl.BlockSpec(memory_space=pltpu.VMEM),
        ],
        out_specs=[
            pl.BlockSpec(memory_space=pltpu.VMEM),
            pl.BlockSpec(memory_space=pltpu.VMEM),
        ],
    )(ishape, roist, scoret, bboxt, gtb, gtl2)
    return out[0][0, 0], out[1][0, 0]


def kernel(image_shape, rois, roi_score, roi_bboxes_txtytwth, gt_bboxes,
           gt_labels):
    gtl2 = gt_labels.astype(jnp.int32).reshape(_NGT, 1)
    return _run(image_shape.astype(jnp.int32),
                rois.astype(jnp.float32).T,
                roi_score.astype(jnp.float32).T,
                roi_bboxes_txtytwth.astype(jnp.float32).T,
                gt_bboxes.astype(jnp.float32), gtl2)
